# Initial kernel scaffold; baseline (speedup 1.0000x reference)
import jax
import jax.numpy as jnp
from jax.experimental import pallas as pl


def kernel(x, n_id, msg, t, edge_index, src_indic, mem, last_update, emb0, emb1, emb2, emb3, emb4, emb5, emb6, emb7, emb8, emb9, W_dir, b_dir, W_edge, b_edge, W_msg, b_msg, W_self):
    raise NotImplementedError("write your pallas kernel here")


# speedup vs baseline: 1.5193x; 1.5193x over previous
import functools

import jax
import jax.numpy as jnp
from jax import lax
from jax.experimental import pallas as pl
from jax.experimental.pallas import tpu as pltpu
from jax.experimental.pallas import tpu_sc as plsc

_NC = 2
_NS = 16
_W = 80
_ROWS_LO = 624
_ROWS_HI = 640


def _node_gather(mem, x16, nidp):
    bp = nidp.shape[0]
    per = bp // (_NC * _NS)
    mesh = plsc.VectorSubcoreMesh(core_axis_name="c", subcore_axis_name="s")

    @functools.partial(
        pl.kernel,
        mesh=mesh,
        out_type=(
            jax.ShapeDtypeStruct((bp, 128), jnp.float32),
            jax.ShapeDtypeStruct((bp, 128), jnp.int32),
        ),
        scratch_types=[
            pltpu.VMEM((_W,), jnp.int32),
            pltpu.VMEM((_W, 128), jnp.float32),
            pltpu.VMEM((_W, 128), jnp.int32),
            pltpu.SemaphoreType.DMA,
            pltpu.SemaphoreType.DMA,
        ],
    )
    def k(mem_hbm, x_hbm, nid_hbm, zm_out, xb_out, nidx, mrows, xrows, s1, s2):
        wid = lax.axis_index("s") * _NC + lax.axis_index("c")

        def win(w, carry):
            base = wid * per + w * _W
            pltpu.sync_copy(nid_hbm.at[pl.ds(base, _W)], nidx)
            cm = pltpu.async_copy(mem_hbm.at[nidx], mrows, s1)
            cx = pltpu.async_copy(x_hbm.at[nidx], xrows, s2)
            cm.wait()
            cx.wait()
            pltpu.sync_copy(mrows, zm_out.at[pl.ds(base, _W)])
            pltpu.sync_copy(xrows, xb_out.at[pl.ds(base, _W)])
            return carry

        lax.fori_loop(0, per // _W, win, 0)

    return k(mem, x16, nidp)


def _edge_sc(zw_flat, mp_flat, src, dst):
    e = src.shape[0]
    b = zw_flat.shape[0] // 2
    wins = e // (_NS * _W)
    per_tile = e // _NS
    rows_per_tile = b // _NS
    mesh = plsc.VectorSubcoreMesh(core_axis_name="c", subcore_axis_name="s")

    @functools.partial(
        pl.kernel,
        mesh=mesh,
        out_type=jax.ShapeDtypeStruct((2 * b, 128), jnp.float32),
        scratch_types=[
            pltpu.VMEM((2, _W), jnp.int32),
            pltpu.VMEM((2, _W), jnp.int32),
            pltpu.VMEM((2, _W, 128), jnp.float32),
            pltpu.VMEM((2, _W, 128), jnp.float32),
            pltpu.VMEM_SHARED((b, 128), jnp.float32),
            pltpu.SemaphoreType.DMA((2,)),
            pltpu.SemaphoreType.DMA((2,)),
        ],
    )
    def k(zw_hbm, mp_hbm, src_hbm, dst_hbm, out_hbm,
          sidx, didx, gbuf, mbuf, agg_sh, gsem, msem):
        cid = lax.axis_index("c")
        tid = lax.axis_index("s")
        rowoff = cid * b
        tbase = tid * per_tile

        def z16(t, carry):
            i = t // 8
            c = (t % 8) * 16
            gbuf[0, i, pl.ds(c, 16)] = jnp.zeros((16,), jnp.float32)
            return carry

        lax.fori_loop(0, _W * 8, z16, 0)

        def zero_rows(base, n):
            full = n // _W
            for r in range(full):
                pltpu.sync_copy(gbuf.at[0], agg_sh.at[pl.ds(base + r * _W, _W)])
            rem = n - full * _W
            if rem:
                pltpu.sync_copy(gbuf.at[0, pl.ds(0, rem)],
                                agg_sh.at[pl.ds(base + full * _W, rem)])

        @pl.when(tid < _NS - 1)
        def _():
            zero_rows(tid * _ROWS_LO, _ROWS_LO)

        @pl.when(tid == _NS - 1)
        def _():
            zero_rows((_NS - 1) * _ROWS_LO, _ROWS_HI)

        plsc.subcore_barrier()

        def issue(slot, w):
            base = tbase + w * _W
            pltpu.sync_copy(src_hbm.at[pl.ds(base, _W)], sidx.at[slot])
            pltpu.sync_copy(dst_hbm.at[pl.ds(base, _W)], didx.at[slot])
            for j in range(_W // 16):
                sl = pl.ds(j * 16, 16)
                sidx[slot, sl] = sidx[slot, sl] + rowoff
            pltpu.async_copy(zw_hbm.at[sidx.at[slot]], gbuf.at[slot],
                             gsem.at[slot])
            pltpu.async_copy(mp_hbm.at[pl.ds(cid * e + base, _W)],
                             mbuf.at[slot], msem.at[slot])

        issue(0, 0)

        def body(kk, carry):
            for slot in range(2):
                w = kk * 2 + slot
                base = tbase + w * _W
                pltpu.make_async_copy(zw_hbm.at[sidx.at[slot]],
                                      gbuf.at[slot], gsem.at[slot]).wait()
                pltpu.make_async_copy(mp_hbm.at[pl.ds(cid * e + base, _W)],
                                      mbuf.at[slot], msem.at[slot]).wait()
                nxt = 1 - slot

                @pl.when(w + 1 < wins)
                def _():
                    issue(nxt, w + 1)

                def comp(t, c2):
                    i = t // 8
                    c = (t % 8) * 16
                    sl = pl.ds(c, 16)
                    gbuf[slot, i, sl] = jnp.maximum(
                        gbuf[slot, i, sl] + mbuf[slot, i, sl], 0.0)
                    return c2

                lax.fori_loop(0, _W * 8, comp, 0)
                pltpu.sync_copy(gbuf.at[slot], agg_sh.at[didx.at[slot]],
                                add=True)
            return carry

        lax.fori_loop(0, wins // 2, body, 0)
        plsc.subcore_barrier()

        @pl.when(tid < _NS - 1)
        def _():
            pltpu.sync_copy(
                agg_sh.at[pl.ds(tid * _ROWS_LO, _ROWS_LO)],
                out_hbm.at[pl.ds(rowoff + tid * _ROWS_LO, _ROWS_LO)])

        @pl.when(tid == _NS - 1)
        def _():
            pltpu.sync_copy(
                agg_sh.at[pl.ds((_NS - 1) * _ROWS_LO, _ROWS_HI)],
                out_hbm.at[pl.ds(rowoff + (_NS - 1) * _ROWS_LO, _ROWS_HI)])

    return k(zw_flat, mp_flat, src, dst)


def _enc_body(zmem_ref, xb_ref, e3_ref, w1_ref, ws_ref, zw_ref, zself_ref):
    xb = xb_ref[...]

    def pick(h):
        sel = xb[:, h:h + 1]
        t = e3_ref[h]
        return jnp.where(sel == 0, t[0:1, :],
                         jnp.where(sel == 1, t[1:2, :], t[2:3, :]))

    g0 = pick(0)
    files = g0 + pick(1) + pick(2) + pick(3) + pick(4)
    proc = g0 + pick(5)
    sock = g0 + pick(6) + pick(7) + pick(8) + pick(9)
    nt = xb[:, 0:1]
    xnew = jnp.where(nt == 0, files, jnp.where(nt == 1, proc, sock))
    z = jnp.concatenate([zmem_ref[...], xnew], axis=1)
    zw = jnp.dot(z, w1_ref[...], preferred_element_type=jnp.float32)
    zw_ref[0] = zw[:, :128]
    zw_ref[1] = zw[:, 128:]
    zself_ref[...] = jnp.dot(z, ws_ref[...], preferred_element_type=jnp.float32)


def _encode_call(zmem, xb, e3, w1, ws):
    bn = zmem.shape[0]
    blk = 1000
    grid = bn // blk
    return pl.pallas_call(
        _enc_body,
        grid=(grid,),
        in_specs=[
            pl.BlockSpec((blk, 128), lambda i: (i, 0)),
            pl.BlockSpec((blk, 128), lambda i: (i, 0)),
            pl.BlockSpec((10, 3, 128), lambda i: (0, 0, 0)),
            pl.BlockSpec((256, 256), lambda i: (0, 0)),
            pl.BlockSpec((256, 256), lambda i: (0, 0)),
        ],
        out_specs=[
            pl.BlockSpec((2, blk, 128), lambda i: (0, i, 0)),
            pl.BlockSpec((blk, 256), lambda i: (i, 0)),
        ],
        out_shape=[
            jax.ShapeDtypeStruct((2, bn, 128), jnp.float32),
            jax.ShapeDtypeStruct((bn, 256), jnp.float32),
        ],
    )(zmem, xb, e3, w1, ws)


def _mproj_body(msga_ref, we_ref, wd_ref, be_ref, bd_ref, bm_ref, w2_ref,
                out_ref):
    wcat = jnp.concatenate([we_ref[...], wd_ref[...]], axis=0)
    wp = jnp.dot(wcat, w2_ref[...], preferred_element_type=jnp.float32)
    cvec = jnp.dot(be_ref[...] + bd_ref[...], w2_ref[...],
                   preferred_element_type=jnp.float32) + bm_ref[...]
    mp = jnp.dot(msga_ref[...], wp, preferred_element_type=jnp.float32) + cvec
    out_ref[0] = mp[:, :128]
    out_ref[1] = mp[:, 128:]


def _mproj_call(msga, we, wd, be2, bd2, bm2, w2):
    e = msga.shape[0]
    blk = 512
    grid = e // blk
    return pl.pallas_call(
        _mproj_body,
        grid=(grid,),
        in_specs=[
            pl.BlockSpec((blk, 17), lambda i: (i, 0)),
            pl.BlockSpec((16, 27), lambda i: (0, 0)),
            pl.BlockSpec((1, 27), lambda i: (0, 0)),
            pl.BlockSpec((1, 27), lambda i: (0, 0)),
            pl.BlockSpec((1, 27), lambda i: (0, 0)),
            pl.BlockSpec((1, 256), lambda i: (0, 0)),
            pl.BlockSpec((27, 256), lambda i: (0, 0)),
        ],
        out_specs=pl.BlockSpec((2, blk, 128), lambda i: (0, i, 0)),
        out_shape=jax.ShapeDtypeStruct((2, e, 128), jnp.float32),
    )(msga, we, wd, be2, bd2, bm2, w2)


def _fin_body(zself_ref, agg_ref, out_ref):
    a = jnp.concatenate([agg_ref[0], agg_ref[1]], axis=1)
    out_ref[...] = jnp.maximum(zself_ref[...] + a, 0.0)


def _final_call(zself, agg2):
    bn = zself.shape[0]
    blk = 1000
    return pl.pallas_call(
        _fin_body,
        grid=(bn // blk,),
        in_specs=[
            pl.BlockSpec((blk, 256), lambda i: (i, 0)),
            pl.BlockSpec((2, blk, 128), lambda i: (0, i, 0)),
        ],
        out_specs=pl.BlockSpec((blk, 256), lambda i: (i, 0)),
        out_shape=jax.ShapeDtypeStruct((bn, 256), jnp.float32),
    )(zself, agg2)


def kernel(x, n_id, msg, t, edge_index, src_indic, mem, last_update,
           emb0, emb1, emb2, emb3, emb4, emb5, emb6, emb7, emb8, emb9,
           W_dir, b_dir, W_edge, b_edge, W_msg, b_msg, W_self):
    b = n_id.shape[0]
    e = msg.shape[0]
    unit = _NC * _NS * _W
    bp = ((b + unit - 1) // unit) * unit

    x16 = jnp.pad(x, ((0, 0), (0, 128 - x.shape[1])))
    nidp = jnp.pad(n_id, (0, bp - b))

    zm_p, xb_p = _node_gather(mem, x16, nidp)
    zmem = zm_p[:b]
    xb = xb_p[:b]

    e3 = jnp.stack([emb0[:3], emb1[:3], emb2[:3], emb3[:3], emb4[:3],
                    emb5[:3], emb6[:3], emb7[:3], emb8[:3], emb9[:3]])
    w1 = W_msg[:256]
    w2 = W_msg[256:]

    zw2, zself = _encode_call(zmem, xb, e3, w1, W_self)
    zw_flat = zw2.reshape(2 * b, 128)

    msga = jnp.concatenate(
        [msg, src_indic.astype(jnp.float32)[:, None]], axis=1)
    mp2 = _mproj_call(msga, W_edge, W_dir, b_edge.reshape(1, 27),
                      b_dir.reshape(1, 27), b_msg.reshape(1, 256), w2)
    mp_flat = mp2.reshape(2 * e, 128)

    agg_flat = _edge_sc(zw_flat, mp_flat, edge_index[0], edge_index[1])
    return _final_call(zself, agg_flat.reshape(2, b, 128))
